# baseline (speedup 1.0000x reference)
import functools

import jax
import jax.numpy as jnp
from jax import lax
from jax.experimental import pallas as pl
from jax.experimental.pallas import tpu as pltpu
from jax.experimental.pallas import tpu_sc as plsc

GROUPS = 4
K = 512
DG = 64
CTX = 7

NC = 2
NS = 16
NW = NC * NS


def _tc_kernel(x_ref, emb_ref, w2_ref, pred_ref, idx_ref, etab_ref):
    tt = x_ref.shape[2]

    @pl.when(pl.program_id(0) == 0)
    def _():
        for g in range(GROUPS):
            etab_ref[g] = jnp.swapaxes(emb_ref[g], 0, 1)
    for p in range(GROUPS // 2):
        rows = slice(p * 2 * DG, (p + 1) * 2 * DG)
        cur = x_ref[0, rows, :]
        cur_t = jnp.swapaxes(cur, 0, 1)
        buf_t = jnp.concatenate(
            [jnp.zeros((8, 2 * DG), jnp.float32), cur_t], axis=0)
        pred_t = jnp.zeros((tt, 2 * DG), jnp.float32)
        for s in range(1, CTX + 1):
            xs = pltpu.roll(buf_t, s, axis=0)[8:, :]
            w = w2_ref[CTX - s, p]
            pred_t = pred_t + lax.dot_general(
                xs, w, (((1,), (1,)), ((), ())),
                preferred_element_type=jnp.float32)
        r_t = cur_t - pred_t
        pred = jnp.swapaxes(pred_t, 0, 1)
        pred_ref[0, rows, :] = pred
        for h in range(2):
            g = 2 * p + h
            rg_t = r_t[:, h * DG:(h + 1) * DG]
            emb = emb_ref[g]
            er = lax.dot_general(emb, rg_t, (((1,), (1,)), ((), ())),
                                 preferred_element_type=jnp.float32)
            half_e2 = 0.5 * jnp.sum(emb * emb, axis=1)[:, None]
            score = er - half_e2
            idx_ref[0, g, :] = jnp.argmax(score, axis=0).astype(jnp.int32)


def _tc_stage(x, embedding, conv_w):
    b, c, t = x.shape
    z = jnp.zeros((CTX, GROUPS // 2, DG, DG), jnp.float32)
    wt = jnp.transpose(conv_w, (3, 0, 1, 2))
    top = jnp.concatenate([wt[:, 0::2], z], axis=-1)
    bot = jnp.concatenate([z, wt[:, 1::2]], axis=-1)
    w2 = jnp.concatenate([top, bot], axis=-2)
    return pl.pallas_call(
        _tc_kernel,
        grid=(b,),
        in_specs=[
            pl.BlockSpec((1, c, t), lambda i: (i, 0, 0)),
            pl.BlockSpec((GROUPS, K, DG), lambda i: (0, 0, 0)),
            pl.BlockSpec((CTX, GROUPS // 2, 2 * DG, 2 * DG),
                         lambda i: (0, 0, 0, 0)),
        ],
        out_specs=[
            pl.BlockSpec((1, c, t), lambda i: (i, 0, 0)),
            pl.BlockSpec((1, GROUPS, t), lambda i: (i, 0, 0)),
            pl.BlockSpec((GROUPS, DG, K), lambda i: (0, 0, 0)),
        ],
        out_shape=[
            jax.ShapeDtypeStruct((b, c, t), jnp.float32),
            jax.ShapeDtypeStruct((b, GROUPS, t), jnp.int32),
            jax.ShapeDtypeStruct((GROUPS, DG, K), jnp.float32),
        ],
        compiler_params=pltpu.CompilerParams(
            dimension_semantics=("arbitrary",),
        ),
    )(x, embedding, w2)


def _sc_stage(pred, idx, etab):
    b, c, t = pred.shape
    rows_per_w = c // NW
    nv = t // 16

    mesh = plsc.VectorSubcoreMesh(core_axis_name="c", subcore_axis_name="s")

    @functools.partial(
        pl.kernel,
        mesh=mesh,
        out_type=jax.ShapeDtypeStruct((b, c, t), jnp.float32),
        scratch_types=[
            pltpu.VMEM((rows_per_w * K,), jnp.float32),
            pltpu.VMEM((t,), jnp.int32),
            pltpu.VMEM((rows_per_w, t), jnp.float32),
            pltpu.VMEM((rows_per_w, t), jnp.float32),
        ],
        compiler_params=pltpu.CompilerParams(needs_layout_passes=False),
    )
    def k(pred_hbm, idx_hbm, etab_hbm, out_hbm, tabs, idxv, predb, outb):
        wid = lax.axis_index("s") * NC + lax.axis_index("c")
        g = wid // (NW // GROUPS)
        oct_ = wid % (NW // GROUPS)
        d0 = oct_ * rows_per_w
        ch0 = g * DG + d0
        pltpu.sync_copy(etab_hbm.at[g, pl.ds(d0 * K, rows_per_w * K)], tabs)
        for bb in range(b):
            pltpu.sync_copy(idx_hbm.at[bb, g], idxv)
            pltpu.sync_copy(pred_hbm.at[bb, pl.ds(ch0, rows_per_w)], predb)
            for j in range(rows_per_w):

                def body(i, _, j=j):
                    sl = pl.ds(pl.multiple_of(i * 16, 16), 16)
                    iv = idxv[sl] + j * K
                    vals = plsc.load_gather(tabs, [iv])
                    outb[j, sl] = predb[j, sl] + vals
                    return _

                lax.fori_loop(0, nv, body, 0)
            pltpu.sync_copy(outb, out_hbm.at[bb, pl.ds(ch0, rows_per_w)])

    return k(pred, idx, etab)


def kernel(x, embedding, conv_w):
    pred, idx, etab = _tc_stage(x, embedding, conv_w)
    return _sc_stage(pred, idx, etab.reshape(GROUPS, DG * K))


# speedup vs baseline: 1.6277x; 1.6277x over previous
import functools

import jax
import jax.numpy as jnp
from jax import lax
from jax.experimental import pallas as pl
from jax.experimental.pallas import tpu as pltpu
from jax.experimental.pallas import tpu_sc as plsc

GROUPS = 4
K = 512
DG = 64
CTX = 7

NC = 2
NS = 16
NW = NC * NS


def _tc_kernel(x_ref, emb_ref, w2_ref, pred_ref, idx_ref, etab_ref):
    tt = x_ref.shape[2]

    @pl.when(pl.program_id(0) == 0)
    def _():
        for g in range(GROUPS):
            etab_ref[g] = jnp.swapaxes(emb_ref[g], 0, 1)
    for p in range(GROUPS // 2):
        rows = slice(p * 2 * DG, (p + 1) * 2 * DG)
        cur = x_ref[0, rows, :]
        cur_t = jnp.swapaxes(cur, 0, 1)
        buf_t = jnp.concatenate(
            [jnp.zeros((8, 2 * DG), jnp.float32), cur_t], axis=0)
        pred_t = jnp.zeros((tt, 2 * DG), jnp.float32)
        for s in range(1, CTX + 1):
            xs = pltpu.roll(buf_t, s, axis=0)[8:, :]
            w = w2_ref[CTX - s, p]
            pred_t = pred_t + lax.dot_general(
                xs, w, (((1,), (1,)), ((), ())),
                preferred_element_type=jnp.float32)
        r_t = cur_t - pred_t
        pred = jnp.swapaxes(pred_t, 0, 1)
        pred_ref[0, rows, :] = pred
        for h in range(2):
            g = 2 * p + h
            rg_t = r_t[:, h * DG:(h + 1) * DG]
            emb = emb_ref[g]
            er = lax.dot_general(emb, rg_t, (((1,), (1,)), ((), ())),
                                 preferred_element_type=jnp.float32)
            half_e2 = 0.5 * jnp.sum(emb * emb, axis=1)[:, None]
            score = er - half_e2
            idx_ref[0, g, :] = jnp.argmax(score, axis=0).astype(jnp.int32)


def _tc_stage(x, embedding, conv_w):
    b, c, t = x.shape
    z = jnp.zeros((CTX, GROUPS // 2, DG, DG), jnp.float32)
    wt = jnp.transpose(conv_w, (3, 0, 1, 2))
    top = jnp.concatenate([wt[:, 0::2], z], axis=-1)
    bot = jnp.concatenate([z, wt[:, 1::2]], axis=-1)
    w2 = jnp.concatenate([top, bot], axis=-2)
    return pl.pallas_call(
        _tc_kernel,
        grid=(b,),
        in_specs=[
            pl.BlockSpec((1, c, t), lambda i: (i, 0, 0)),
            pl.BlockSpec((GROUPS, K, DG), lambda i: (0, 0, 0)),
            pl.BlockSpec((CTX, GROUPS // 2, 2 * DG, 2 * DG),
                         lambda i: (0, 0, 0, 0)),
        ],
        out_specs=[
            pl.BlockSpec((1, c, t), lambda i: (i, 0, 0)),
            pl.BlockSpec((1, GROUPS, t), lambda i: (i, 0, 0)),
            pl.BlockSpec((GROUPS, DG, K), lambda i: (0, 0, 0)),
        ],
        out_shape=[
            jax.ShapeDtypeStruct((b, c, t), jnp.float32),
            jax.ShapeDtypeStruct((b, GROUPS, t), jnp.int32),
            jax.ShapeDtypeStruct((GROUPS, DG, K), jnp.float32),
        ],
        compiler_params=pltpu.CompilerParams(
            dimension_semantics=("arbitrary",),
        ),
    )(x, embedding, w2)


def _sc_stage(pred, idx, etab):
    b, c, t = pred.shape
    rows_per_w = c // NW
    nv = t // 16

    mesh = plsc.VectorSubcoreMesh(core_axis_name="c", subcore_axis_name="s")

    @functools.partial(
        pl.kernel,
        mesh=mesh,
        out_type=jax.ShapeDtypeStruct((b, c, t), jnp.float32),
        scratch_types=[
            pltpu.VMEM((rows_per_w * K,), jnp.float32),
            pltpu.VMEM((t,), jnp.int32),
            pltpu.VMEM((rows_per_w, t), jnp.float32),
            pltpu.VMEM((rows_per_w, t), jnp.float32),
        ],
        compiler_params=pltpu.CompilerParams(needs_layout_passes=False),
    )
    def k(pred_hbm, idx_hbm, etab_hbm, out_hbm, tabs, idxv, predb, outb):
        wid = lax.axis_index("s") * NC + lax.axis_index("c")
        g = wid // (NW // GROUPS)
        oct_ = wid % (NW // GROUPS)
        d0 = oct_ * rows_per_w
        ch0 = g * DG + d0
        pltpu.sync_copy(etab_hbm.at[g, pl.ds(d0 * K, rows_per_w * K)], tabs)
        for bb in range(b):
            pltpu.sync_copy(idx_hbm.at[bb, g], idxv)
            pltpu.sync_copy(pred_hbm.at[bb, pl.ds(ch0, rows_per_w)], predb)

            @plsc.parallel_loop(0, nv, unroll=8)
            def body(i):
                sl = pl.ds(pl.multiple_of(i * 16, 16), 16)
                iv = idxv[sl]
                for j in range(rows_per_w):
                    vals = plsc.load_gather(tabs, [iv + j * K])
                    outb[j, sl] = predb[j, sl] + vals

            pltpu.sync_copy(outb, out_hbm.at[bb, pl.ds(ch0, rows_per_w)])

    return k(pred, idx, etab)


def kernel(x, embedding, conv_w):
    pred, idx, etab = _tc_stage(x, embedding, conv_w)
    return _sc_stage(pred, idx, etab.reshape(GROUPS, DG * K))


# speedup vs baseline: 1.9332x; 1.1877x over previous
import functools

import jax
import jax.numpy as jnp
from jax import lax
from jax.experimental import pallas as pl
from jax.experimental.pallas import tpu as pltpu
from jax.experimental.pallas import tpu_sc as plsc

GROUPS = 4
K = 512
DG = 64
CTX = 7

NC = 2
NS = 16
NW = NC * NS


def _tc_kernel(x_ref, emb_ref, w2_ref, pred_ref, idx_ref, etab_ref):
    tt = x_ref.shape[2]

    @pl.when(pl.program_id(0) == 0)
    def _():
        for g in range(GROUPS):
            etab_ref[g] = jnp.swapaxes(emb_ref[g], 0, 1)
    for p in range(GROUPS // 2):
        rows = slice(p * 2 * DG, (p + 1) * 2 * DG)
        cur = x_ref[0, rows, :]
        cur_t = jnp.swapaxes(cur, 0, 1)
        buf_t = jnp.concatenate(
            [jnp.zeros((8, 2 * DG), jnp.float32), cur_t], axis=0)
        pred_t = jnp.zeros((tt, 2 * DG), jnp.float32)
        for s in range(1, CTX + 1):
            xs = pltpu.roll(buf_t, s, axis=0)[8:, :]
            w = w2_ref[CTX - s, p]
            pred_t = pred_t + lax.dot_general(
                xs, w, (((1,), (1,)), ((), ())),
                preferred_element_type=jnp.float32)
        r_t = cur_t - pred_t
        pred = jnp.swapaxes(pred_t, 0, 1)
        pred_ref[0, rows, :] = pred
        for h in range(2):
            g = 2 * p + h
            rg_t = r_t[:, h * DG:(h + 1) * DG]
            emb = emb_ref[g]
            er = lax.dot_general(emb, rg_t, (((1,), (1,)), ((), ())),
                                 preferred_element_type=jnp.float32)
            half_e2 = 0.5 * jnp.sum(emb * emb, axis=1)[:, None]
            score = er - half_e2
            idx_ref[0, g, :] = jnp.argmax(score, axis=0).astype(jnp.int32)


def _tc_stage(x, embedding, conv_w):
    b, c, t = x.shape
    z = jnp.zeros((CTX, GROUPS // 2, DG, DG), jnp.float32)
    wt = jnp.transpose(conv_w, (3, 0, 1, 2))
    top = jnp.concatenate([wt[:, 0::2], z], axis=-1)
    bot = jnp.concatenate([z, wt[:, 1::2]], axis=-1)
    w2 = jnp.concatenate([top, bot], axis=-2)
    return pl.pallas_call(
        _tc_kernel,
        grid=(b,),
        in_specs=[
            pl.BlockSpec((1, c, t), lambda i: (i, 0, 0)),
            pl.BlockSpec((GROUPS, K, DG), lambda i: (0, 0, 0)),
            pl.BlockSpec((CTX, GROUPS // 2, 2 * DG, 2 * DG),
                         lambda i: (0, 0, 0, 0)),
        ],
        out_specs=[
            pl.BlockSpec((1, c, t), lambda i: (i, 0, 0)),
            pl.BlockSpec((1, GROUPS, t), lambda i: (i, 0, 0)),
            pl.BlockSpec((GROUPS, DG, K), lambda i: (0, 0, 0)),
        ],
        out_shape=[
            jax.ShapeDtypeStruct((b, c, t), jnp.float32),
            jax.ShapeDtypeStruct((b, GROUPS, t), jnp.int32),
            jax.ShapeDtypeStruct((GROUPS, DG, K), jnp.float32),
        ],
        compiler_params=pltpu.CompilerParams(
            dimension_semantics=("arbitrary",),
        ),
    )(x, embedding, w2)


def _sc_stage(pred, idx, etab):
    b, c, t = pred.shape
    rows_per_w = c // NW
    nv = t // 16

    mesh = plsc.VectorSubcoreMesh(core_axis_name="c", subcore_axis_name="s")

    @functools.partial(
        pl.kernel,
        mesh=mesh,
        out_type=jax.ShapeDtypeStruct((b, c, t), jnp.float32),
        scratch_types=[
            pltpu.VMEM((rows_per_w * K,), jnp.float32),
            pltpu.VMEM((b, t), jnp.int32),
            pltpu.VMEM((2, rows_per_w, t), jnp.float32),
            pltpu.VMEM((2, rows_per_w, t), jnp.float32),
            pltpu.SemaphoreType.DMA,
            pltpu.SemaphoreType.DMA,
            pltpu.SemaphoreType.DMA,
            pltpu.SemaphoreType.DMA,
        ],
        compiler_params=pltpu.CompilerParams(needs_layout_passes=False),
    )
    def k(pred_hbm, idx_hbm, etab_hbm, out_hbm, tabs, idxv, predb, outb,
          isem0, isem1, osem0, osem1):
        wid = lax.axis_index("s") * NC + lax.axis_index("c")
        g = wid // (NW // GROUPS)
        oct_ = wid % (NW // GROUPS)
        d0 = oct_ * rows_per_w
        ch0 = g * DG + d0
        isems = [isem0, isem1]
        osems = [osem0, osem1]
        h_in = [None, None]
        h_out = [None, None]
        h_in[0] = pltpu.async_copy(
            pred_hbm.at[0, pl.ds(ch0, rows_per_w)], predb.at[0], isems[0])
        pltpu.sync_copy(etab_hbm.at[g, pl.ds(d0 * K, rows_per_w * K)], tabs)
        pltpu.sync_copy(idx_hbm.at[:, g], idxv)
        for bb in range(b):
            s = bb % 2
            if bb + 1 < b:
                h_in[1 - s] = pltpu.async_copy(
                    pred_hbm.at[bb + 1, pl.ds(ch0, rows_per_w)],
                    predb.at[1 - s], isems[1 - s])
            h_in[s].wait()
            if h_out[s] is not None:
                h_out[s].wait()

            @plsc.parallel_loop(0, nv, unroll=8)
            def body(i, s=s, bb=bb):
                sl = pl.ds(pl.multiple_of(i * 16, 16), 16)
                iv = idxv[bb, sl]
                for j in range(rows_per_w):
                    vals = plsc.load_gather(tabs, [iv + j * K])
                    outb[s, j, sl] = predb[s, j, sl] + vals

            h_out[s] = pltpu.async_copy(
                outb.at[s], out_hbm.at[bb, pl.ds(ch0, rows_per_w)], osems[s])
        h_out[0].wait()
        h_out[1].wait()

    return k(pred, idx, etab)


def kernel(x, embedding, conv_w):
    pred, idx, etab = _tc_stage(x, embedding, conv_w)
    return _sc_stage(pred, idx, etab.reshape(GROUPS, DG * K))


# speedup vs baseline: 1.9388x; 1.0029x over previous
import functools

import jax
import jax.numpy as jnp
from jax import lax
from jax.experimental import pallas as pl
from jax.experimental.pallas import tpu as pltpu
from jax.experimental.pallas import tpu_sc as plsc

GROUPS = 4
K = 512
DG = 64
CTX = 7

NC = 2
NS = 16
NW = NC * NS


def _tc_kernel(x_ref, emb_ref, w2_ref, pred_ref, idx_ref, etab_ref):
    tt = x_ref.shape[2]

    @pl.when(pl.program_id(0) == 0)
    def _():
        for g in range(GROUPS):
            etab_ref[g] = jnp.swapaxes(emb_ref[g], 0, 1)
    for p in range(GROUPS // 2):
        rows = slice(p * 2 * DG, (p + 1) * 2 * DG)
        cur = x_ref[0, rows, :]
        cur_t = jnp.swapaxes(cur, 0, 1)
        buf_t = jnp.concatenate(
            [jnp.zeros((8, 2 * DG), jnp.float32), cur_t], axis=0)
        pred_t = jnp.zeros((tt, 2 * DG), jnp.float32)
        for s in range(1, CTX + 1):
            xs = pltpu.roll(buf_t, s, axis=0)[8:, :]
            w = w2_ref[CTX - s, p]
            pred_t = pred_t + lax.dot_general(
                xs, w, (((1,), (1,)), ((), ())),
                preferred_element_type=jnp.float32)
        r_t = cur_t - pred_t
        pred = jnp.swapaxes(pred_t, 0, 1)
        pred_ref[0, rows, :] = pred
        for h in range(2):
            g = 2 * p + h
            rg_t = r_t[:, h * DG:(h + 1) * DG]
            emb = emb_ref[g]
            er = lax.dot_general(emb, rg_t, (((1,), (1,)), ((), ())),
                                 preferred_element_type=jnp.float32)
            half_e2 = 0.5 * jnp.sum(emb * emb, axis=1)[:, None]
            score = er - half_e2
            idx_ref[0, g, :] = jnp.argmax(score, axis=0).astype(jnp.int32)


def _tc_stage(x, embedding, conv_w):
    b, c, t = x.shape
    z = jnp.zeros((CTX, GROUPS // 2, DG, DG), jnp.float32)
    wt = jnp.transpose(conv_w, (3, 0, 1, 2))
    top = jnp.concatenate([wt[:, 0::2], z], axis=-1)
    bot = jnp.concatenate([z, wt[:, 1::2]], axis=-1)
    w2 = jnp.concatenate([top, bot], axis=-2)
    return pl.pallas_call(
        _tc_kernel,
        grid=(b,),
        in_specs=[
            pl.BlockSpec((1, c, t), lambda i: (i, 0, 0)),
            pl.BlockSpec((GROUPS, K, DG), lambda i: (0, 0, 0)),
            pl.BlockSpec((CTX, GROUPS // 2, 2 * DG, 2 * DG),
                         lambda i: (0, 0, 0, 0)),
        ],
        out_specs=[
            pl.BlockSpec((1, c, t), lambda i: (i, 0, 0)),
            pl.BlockSpec((1, GROUPS, t), lambda i: (i, 0, 0)),
            pl.BlockSpec((GROUPS, DG, K), lambda i: (0, 0, 0)),
        ],
        out_shape=[
            jax.ShapeDtypeStruct((b, c, t), jnp.float32),
            jax.ShapeDtypeStruct((b, GROUPS, t), jnp.int32),
            jax.ShapeDtypeStruct((GROUPS, DG, K), jnp.float32),
        ],
        compiler_params=pltpu.CompilerParams(
            dimension_semantics=("arbitrary",),
        ),
    )(x, embedding, w2)


def _sc_stage(pred, idx, etab):
    b, c, t = pred.shape
    rows_per_w = c // NW
    nv = t // 16

    mesh = plsc.VectorSubcoreMesh(core_axis_name="c", subcore_axis_name="s")

    @functools.partial(
        pl.kernel,
        mesh=mesh,
        out_type=jax.ShapeDtypeStruct((b, c, t), jnp.float32),
        scratch_types=[
            pltpu.VMEM((rows_per_w * K,), jnp.float32),
            pltpu.VMEM((b, t), jnp.int32),
            pltpu.VMEM((2, rows_per_w, t), jnp.float32),
            pltpu.VMEM((2, rows_per_w, t), jnp.float32),
            pltpu.SemaphoreType.DMA,
            pltpu.SemaphoreType.DMA,
            pltpu.SemaphoreType.DMA,
            pltpu.SemaphoreType.DMA,
        ],
        compiler_params=pltpu.CompilerParams(needs_layout_passes=False),
    )
    def k(pred_hbm, idx_hbm, etab_hbm, out_hbm, tabs, idxv, predb, outb,
          isem0, isem1, osem0, osem1):
        wid = lax.axis_index("s") * NC + lax.axis_index("c")
        g = wid // (NW // GROUPS)
        oct_ = wid % (NW // GROUPS)
        d0 = oct_ * rows_per_w
        ch0 = g * DG + d0
        isems = [isem0, isem1]
        osems = [osem0, osem1]
        h_in = [None, None]
        h_out = [None, None]
        h_in[0] = pltpu.async_copy(
            pred_hbm.at[0, pl.ds(ch0, rows_per_w)], predb.at[0], isems[0])
        pltpu.sync_copy(etab_hbm.at[g, pl.ds(d0 * K, rows_per_w * K)], tabs)
        pltpu.sync_copy(idx_hbm.at[:, g], idxv)
        for bb in range(b):
            s = bb % 2
            if bb + 1 < b:
                h_in[1 - s] = pltpu.async_copy(
                    pred_hbm.at[bb + 1, pl.ds(ch0, rows_per_w)],
                    predb.at[1 - s], isems[1 - s])
            h_in[s].wait()
            if h_out[s] is not None:
                h_out[s].wait()

            @plsc.parallel_loop(0, nv, unroll=8)
            def body(i, s=s, bb=bb):
                sl = pl.ds(pl.multiple_of(i * 16, 16), 16)
                iv = idxv[bb, sl]
                for j in range(rows_per_w):
                    vals = plsc.load_gather(tabs.at[pl.ds(j * K, K)], [iv])
                    outb[s, j, sl] = predb[s, j, sl] + vals

            h_out[s] = pltpu.async_copy(
                outb.at[s], out_hbm.at[bb, pl.ds(ch0, rows_per_w)], osems[s])
        h_out[0].wait()
        h_out[1].wait()

    return k(pred, idx, etab)


def kernel(x, embedding, conv_w):
    pred, idx, etab = _tc_stage(x, embedding, conv_w)
    return _sc_stage(pred, idx, etab.reshape(GROUPS, DG * K))
